# baseline (speedup 1.0000x reference)
import functools

import jax
import jax.numpy as jnp
from jax import lax
from jax.experimental import pallas as pl
from jax.experimental.pallas import tpu as pltpu
from jax.experimental.pallas import tpu_sc as plsc

_B = 16384
_D = 64
_NC = 2
_NS = 16
_NW = _NC * _NS
_BPW = _B // _NW
_CHUNK = 128
_NCH = _BPW // _CHUNK
_LANES = 16


def _sgns_body(c_hbm, w_hbm, ctab_hbm, wtab_hbm, out_hbm,
               cidx_v, widx_v, crows_v, wrows_v, res_v, sem):
    wid = lax.axis_index("s") * _NC + lax.axis_index("c")
    base = wid * _BPW

    for j in range(_NCH):
        pltpu.sync_copy(c_hbm.at[pl.ds(base + j * _CHUNK, _CHUNK)], cidx_v.at[j])
        pltpu.sync_copy(w_hbm.at[pl.ds(base + j * _CHUNK, _CHUNK)], widx_v.at[j])

    copies = []
    for j in range(_NCH):
        copies.append(pltpu.async_copy(
            ctab_hbm.at[cidx_v.at[j]], crows_v.at[pl.ds(j * _CHUNK, _CHUNK)], sem))
        copies.append(pltpu.async_copy(
            wtab_hbm.at[widx_v.at[j]], wrows_v.at[pl.ds(j * _CHUNK, _CHUNK)], sem))
    for cp in copies:
        cp.wait()

    lane = lax.iota(jnp.int32, _LANES)
    def g_body(g, carry):
        row = (g * _LANES + lane).astype(jnp.int32)
        acc = jnp.zeros((_LANES,), jnp.float32)
        for d in range(_D):
            col = jnp.full((_LANES,), d, jnp.int32)
            a = plsc.load_gather(crows_v, [row, col])
            b = plsc.load_gather(wrows_v, [row, col])
            acc = acc + a * b
        res_v[pl.ds(g * _LANES, _LANES)] = 1.0 / (1.0 + jnp.exp(-acc))
        return carry

    lax.fori_loop(0, _BPW // _LANES, g_body, 0)

    pltpu.sync_copy(res_v, out_hbm.at[pl.ds(base, _BPW)])


@jax.jit
def kernel(c, w, c_table, w_table):
    mesh = plsc.VectorSubcoreMesh(core_axis_name="c", subcore_axis_name="s")
    f = pl.kernel(
        _sgns_body,
        out_type=jax.ShapeDtypeStruct((_B,), jnp.float32),
        mesh=mesh,
        scratch_types=[
            pltpu.VMEM((_NCH, _CHUNK), jnp.int32),
            pltpu.VMEM((_NCH, _CHUNK), jnp.int32),
            pltpu.VMEM((_BPW, _D), jnp.float32),
            pltpu.VMEM((_BPW, _D), jnp.float32),
            pltpu.VMEM((_BPW,), jnp.float32),
            pltpu.SemaphoreType.DMA,
        ],
        compiler_params=pltpu.CompilerParams(
            needs_layout_passes=False, use_tc_tiling_on_sc=False),
    )
    return f(c.astype(jnp.int32), w.astype(jnp.int32), c_table, w_table)


# speedup vs baseline: 1.0625x; 1.0625x over previous
import jax
import jax.numpy as jnp
from jax import lax
from jax.experimental import pallas as pl
from jax.experimental.pallas import tpu as pltpu
from jax.experimental.pallas import tpu_sc as plsc

_B = 16384
_D = 64
_NC = 2
_NS = 16
_NW = _NC * _NS
_BPW = _B // _NW
_CHUNK = 128
_HALF = 256
_NCH = _HALF // _CHUNK
_LANES = 16
_PADD = 128
_VOCAB = 1000000


def _sgns_body(c_hbm, w_hbm, ctab_hbm, wtab_hbm, out_hbm,
               cidx_v, widx_v, crows_v, wrows_v, res_v, sem):
    wid = lax.axis_index("s") * _NC + lax.axis_index("c")
    base = wid * _BPW
    lane = lax.iota(jnp.int32, _LANES)

    for half in range(_BPW // _HALF):
        hbase = base + half * _HALF

        for j in range(_NCH):
            pltpu.sync_copy(c_hbm.at[pl.ds(hbase + j * _CHUNK, _CHUNK)],
                            cidx_v.at[j])
            pltpu.sync_copy(w_hbm.at[pl.ds(hbase + j * _CHUNK, _CHUNK)],
                            widx_v.at[j])

        copies = []
        for j in range(_NCH):
            copies.append(pltpu.async_copy(
                ctab_hbm.at[cidx_v.at[j]],
                crows_v.at[pl.ds(j * _CHUNK, _CHUNK)], sem))
            copies.append(pltpu.async_copy(
                wtab_hbm.at[widx_v.at[j]],
                wrows_v.at[pl.ds(j * _CHUNK, _CHUNK)], sem))
        for cp in copies:
            cp.wait()

        def g_body(g, carry):
            row = (g * _LANES + lane).astype(jnp.int32)
            acc = jnp.zeros((_LANES,), jnp.float32)
            for d in range(_D):
                col = jnp.full((_LANES,), d, jnp.int32)
                a = plsc.load_gather(crows_v, [row, col])
                b = plsc.load_gather(wrows_v, [row, col])
                acc = acc + a * b
            res_v[pl.ds(half * _HALF + g * _LANES, _LANES)] = (
                1.0 / (1.0 + jnp.exp(-acc)))
            return carry

        lax.fori_loop(0, _HALF // _LANES, g_body, 0)

    pltpu.sync_copy(res_v, out_hbm.at[pl.ds(base, _BPW)])


@jax.jit
def kernel(c, w, c_table, w_table):
    mesh = plsc.VectorSubcoreMesh(core_axis_name="c", subcore_axis_name="s")
    f = pl.kernel(
        _sgns_body,
        out_type=jax.ShapeDtypeStruct((_B,), jnp.float32),
        mesh=mesh,
        scratch_types=[
            pltpu.VMEM((_NCH, _CHUNK), jnp.int32),
            pltpu.VMEM((_NCH, _CHUNK), jnp.int32),
            pltpu.VMEM((_HALF, _PADD), jnp.float32),
            pltpu.VMEM((_HALF, _PADD), jnp.float32),
            pltpu.VMEM((_BPW,), jnp.float32),
            pltpu.SemaphoreType.DMA,
        ],
        compiler_params=pltpu.CompilerParams(
            needs_layout_passes=False, use_tc_tiling_on_sc=True),
    )
    ct = jnp.pad(c_table, ((0, 0), (0, _PADD - _D)))
    wt = jnp.pad(w_table, ((0, 0), (0, _PADD - _D)))
    return f(c.astype(jnp.int32), w.astype(jnp.int32), ct, wt)
